# baseline (speedup 1.0000x reference)
import functools

import jax
import jax.numpy as jnp
from jax import lax
from jax.experimental import pallas as pl
from jax.experimental.pallas import tpu as pltpu
from jax.experimental.pallas import tpu_sc as plsc

N = 10000
F = 128
E = 320000
LEVELS = 4

NC = 2
NS = 16
NW = NC * NS

C = 64
EROWS = E // 64
PROWS = 5120
PER_TILE = PROWS * 64 // NW
TROWS = PROWS // NW
SB = 2048
SBR = SB // 64
NSB = PER_TILE // SB
SBU = 4
ACC_N = 10240
ROWS_PER_TILE = ACC_N // NS
DUMMY0 = N
XTAIL = N - 15 * ROWS_PER_TILE
ZTAIL = ACC_N - N

NB = 5
BLK = N // NB


def _transform_body(x_ref, lv_ref, w_ref, b_ref, y_ref):
    x = x_ref[...]
    lv = lv_ref[0, 0, :]
    acc = jnp.zeros_like(x)
    for l in range(LEVELS):
        m = (lv == l).astype(jnp.float32)[:, None]
        xw = lax.dot_general(x * m, w_ref[l], (((1,), (1,)), ((), ())),
                             preferred_element_type=jnp.float32)
        acc = acc + xw + m * b_ref[l][None, :]
    y_ref[...] = 0.2 * acc


def _merge_body(a_ref, o_ref):
    o_ref[...] = a_ref[0] + a_ref[1]


def _pack_body(e_ref, o_ref):
    x = e_ref[...]
    o_ref[...] = lax.shift_left(x, 16) | (pltpu.roll(x, 127, 1) & 0xFFFF)


def _sc_body(y_hbm, ed_hbm, lv_hbm, x_hbm, zin_hbm, out_hbm,
             lv_v, sed_a, sed_b, gcomb_v,
             gidx_a, gidx_b, sidx_a, sidx_b, rows_a, rows_b, acc_sh,
             sem_init, sem_lv, sem_sa, sem_sb, sem_a, sem_b):
    c = lax.axis_index("c")
    s = lax.axis_index("s")
    wid = s * NC + c
    rbase = wid * TROWS

    @pl.when((c == 0) & (s < NS - 1))
    def _init0():
        pltpu.async_copy(x_hbm.at[pl.ds(s * ROWS_PER_TILE, ROWS_PER_TILE)],
                         acc_sh.at[pl.ds(s * ROWS_PER_TILE, ROWS_PER_TILE)],
                         sem_init)

    @pl.when((c == 0) & (s == NS - 1))
    def _init0t():
        pltpu.async_copy(x_hbm.at[pl.ds(15 * ROWS_PER_TILE, XTAIL)],
                         acc_sh.at[pl.ds(15 * ROWS_PER_TILE, XTAIL)],
                         sem_init)
        pltpu.async_copy(zin_hbm.at[pl.ds(0, ZTAIL)],
                         acc_sh.at[pl.ds(N, ZTAIL)], sem_init)

    @pl.when(c == 1)
    def _init1():
        pltpu.async_copy(zin_hbm,
                         acc_sh.at[pl.ds(s * ROWS_PER_TILE, ROWS_PER_TILE)],
                         sem_init)

    pltpu.async_copy(lv_hbm, lv_v, sem_lv)
    pltpu.async_copy(ed_hbm.at[pl.ds(rbase, SBR)], sed_a, sem_sa)

    pltpu.make_async_copy(lv_hbm, lv_v, sem_lv).wait()
    iota2 = lax.iota(jnp.int32, 16) * 2

    def compact_block(sed, cnt):
        def cvec(v, cnt):
            for u in range(SBU):
                off = (v * SBU + u) * 16
                p = off * 2 + iota2
                packed = plsc.load_gather(sed, [lax.shift_right_logical(p, 7),
                                                p & 127])
                srcs = lax.shift_right_logical(packed, 16)
                dsts = packed & 0xFFFF
                ll = plsc.load_gather(lv_v, [srcs])
                hl = plsc.load_gather(lv_v, [dsts])
                valid = (ll >= 0) & (ll < LEVELS) & (hl > ll)
                plsc.store_compressed(gcomb_v.at[pl.ds(cnt, 16)], packed,
                                      mask=valid)
                cnt = cnt + plsc.all_reduce_population_count(valid)[0]
            return cnt
        return lax.fori_loop(0, SB // (16 * SBU), cvec, cnt)

    cnt = jnp.int32(0)
    for b in range(NSB):
        cur, cur_sem = (sed_a, sem_sa) if b % 2 == 0 else (sed_b, sem_sb)
        nxt, nxt_sem = (sed_b, sem_sb) if b % 2 == 0 else (sed_a, sem_sa)
        if b + 1 < NSB:
            pltpu.async_copy(ed_hbm.at[pl.ds(rbase + (b + 1) * SBR, SBR)],
                             nxt, nxt_sem)
        pltpu.make_async_copy(ed_hbm.at[pl.ds(0, SBR)], cur, cur_sem).wait()
        cnt = compact_block(cur, cnt)

    for v in range(C // 16):
        dummy = DUMMY0 + v * 16 + lax.iota(jnp.int32, 16)
        gcomb_v[pl.ds(cnt + v * 16, 16)] = dummy

    nch = (cnt + C - 1) // C

    @pl.when((c == 0) & (s < NS - 1))
    def _init0_wait():
        pltpu.make_async_copy(
            x_hbm.at[pl.ds(s * ROWS_PER_TILE, ROWS_PER_TILE)],
            acc_sh.at[pl.ds(s * ROWS_PER_TILE, ROWS_PER_TILE)],
            sem_init).wait()

    @pl.when((c == 0) & (s == NS - 1))
    def _init0t_wait():
        pltpu.make_async_copy(x_hbm.at[pl.ds(15 * ROWS_PER_TILE, XTAIL)],
                              acc_sh.at[pl.ds(15 * ROWS_PER_TILE, XTAIL)],
                              sem_init).wait()
        pltpu.make_async_copy(zin_hbm.at[pl.ds(0, ZTAIL)],
                              acc_sh.at[pl.ds(N, ZTAIL)], sem_init).wait()

    @pl.when(c == 1)
    def _init1_wait():
        pltpu.make_async_copy(
            zin_hbm,
            acc_sh.at[pl.ds(s * ROWS_PER_TILE, ROWS_PER_TILE)],
            sem_init).wait()

    plsc.subcore_barrier()

    def fill_idx(j, gidx, sidx):
        for v in range(C // 16):
            packed = gcomb_v[pl.ds(j * C + v * 16, 16)]
            gidx[pl.ds(v * 16, 16)] = lax.shift_right_logical(packed, 16)
            sidx[pl.ds(v * 16, 16)] = packed & 0xFFFF

    def start_gather(rows, gidx, sem):
        return pltpu.async_copy(y_hbm.at[gidx], rows, sem)

    @pl.when(nch > 0)
    def _prologue():
        fill_idx(0, gidx_a, sidx_a)
        start_gather(rows_a, gidx_a, sem_a)

    def pair(p, carry):
        j0 = 2 * p
        j1 = j0 + 1

        @pl.when(j1 < nch)
        def _startb():
            fill_idx(j1, gidx_b, sidx_b)
            start_gather(rows_b, gidx_b, sem_b)

        pltpu.make_async_copy(y_hbm.at[gidx_a], rows_a, sem_a).wait()
        pltpu.sync_copy(rows_a, acc_sh.at[sidx_a], add=True)

        @pl.when(j0 + 2 < nch)
        def _starta():
            fill_idx(j0 + 2, gidx_a, sidx_a)
            start_gather(rows_a, gidx_a, sem_a)

        @pl.when(j1 < nch)
        def _drainb():
            pltpu.make_async_copy(y_hbm.at[gidx_b], rows_b, sem_b).wait()
            pltpu.sync_copy(rows_b, acc_sh.at[sidx_b], add=True)

        return carry

    lax.fori_loop(0, (nch + 1) // 2, pair, jnp.int32(0))
    plsc.subcore_barrier()

    pltpu.sync_copy(acc_sh.at[pl.ds(s * ROWS_PER_TILE, ROWS_PER_TILE)],
                    out_hbm.at[c, pl.ds(s * ROWS_PER_TILE, ROWS_PER_TILE)])


_sc_edges = functools.partial(
    pl.kernel,
    out_type=jax.ShapeDtypeStruct((NC, ACC_N, F), jnp.float32),
    mesh=plsc.VectorSubcoreMesh(core_axis_name="c", subcore_axis_name="s"),
    compiler_params=pltpu.CompilerParams(needs_layout_passes=False),
    scratch_types=[
        pltpu.VMEM((N,), jnp.int32),
        pltpu.VMEM((SBR, 128), jnp.int32),
        pltpu.VMEM((SBR, 128), jnp.int32),
        pltpu.VMEM((PER_TILE + C,), jnp.int32),
        pltpu.VMEM((C,), jnp.int32),
        pltpu.VMEM((C,), jnp.int32),
        pltpu.VMEM((C,), jnp.int32),
        pltpu.VMEM((C,), jnp.int32),
        pltpu.VMEM((C, F), jnp.float32),
        pltpu.VMEM((C, F), jnp.float32),
        pltpu.VMEM_SHARED((ACC_N, F), jnp.float32),
        pltpu.SemaphoreType.DMA,
        pltpu.SemaphoreType.DMA,
        pltpu.SemaphoreType.DMA,
        pltpu.SemaphoreType.DMA,
        pltpu.SemaphoreType.DMA,
        pltpu.SemaphoreType.DMA,
    ],
)(_sc_body)


def kernel(node_features, hierarchy_edges, hierarchy_levels, level_weights, level_biases):
    ep = jnp.pad(hierarchy_edges.reshape(EROWS, 128),
                 ((0, PROWS - EROWS), (0, 0)))
    pg = pl.pallas_call(
        _pack_body,
        grid=(5,),
        in_specs=[pl.BlockSpec((PROWS // 5, 128), lambda i: (i, 0))],
        out_specs=pl.BlockSpec((PROWS // 5, 128), lambda i: (i, 0)),
        out_shape=jax.ShapeDtypeStruct((PROWS, 128), jnp.int32),
    )(ep)

    lv3 = hierarchy_levels.reshape(NB, 1, BLK)

    y = pl.pallas_call(
        _transform_body,
        grid=(NB,),
        in_specs=[
            pl.BlockSpec((BLK, F), lambda i: (i, 0)),
            pl.BlockSpec((1, 1, BLK), lambda i: (i, 0, 0)),
            pl.BlockSpec((LEVELS, F, F), lambda i: (0, 0, 0)),
            pl.BlockSpec((LEVELS, F), lambda i: (0, 0)),
        ],
        out_specs=pl.BlockSpec((BLK, F), lambda i: (i, 0)),
        out_shape=jax.ShapeDtypeStruct((N, F), jnp.float32),
    )(node_features, lv3, level_weights, level_biases)

    zin = jnp.zeros((ROWS_PER_TILE, F), jnp.float32)
    parts = _sc_edges(y, pg, hierarchy_levels, node_features, zin)

    out = pl.pallas_call(
        _merge_body,
        grid=(NB,),
        in_specs=[
            pl.BlockSpec((NC, BLK, F), lambda i: (0, i, 0)),
        ],
        out_specs=pl.BlockSpec((BLK, F), lambda i: (i, 0)),
        out_shape=jax.ShapeDtypeStruct((N, F), jnp.float32),
    )(parts)
    return out


# speedup vs baseline: 2.0863x; 2.0863x over previous
import functools

import jax
import jax.numpy as jnp
from jax import lax
from jax.experimental import pallas as pl
from jax.experimental.pallas import tpu as pltpu
from jax.experimental.pallas import tpu_sc as plsc

N = 10000
F = 128
E = 320000
LEVELS = 4

NC = 2
NS = 16
NW = NC * NS

C = 64
PER_TILE = E // NW
SB = 2000
NSB = PER_TILE // SB
SBU = 5
ACC_N = 10240
ROWS_PER_TILE = ACC_N // NS
DUMMY0 = N
XTAIL = N - 15 * ROWS_PER_TILE
ZTAIL = ACC_N - N

NB = 5
BLK = N // NB


def _transform_body(x_ref, lv_ref, w_ref, b_ref, y_ref):
    x = x_ref[...]
    lv = lv_ref[0, 0, :]
    acc = jnp.zeros_like(x)
    for l in range(LEVELS):
        m = (lv == l).astype(jnp.float32)[:, None]
        xw = lax.dot_general(x * m, w_ref[l], (((1,), (1,)), ((), ())),
                             preferred_element_type=jnp.float32)
        acc = acc + xw + m * b_ref[l][None, :]
    y_ref[...] = 0.2 * acc


def _merge_body(a_ref, o_ref):
    o_ref[...] = a_ref[0] + a_ref[1]


def _sc_body(y_hbm, ed_hbm, lv_hbm, x_hbm, zin_hbm, out_hbm,
             lv_v, sed_a, sed_b, gcomb_v,
             gidx_a, gidx_b, sidx_a, sidx_b, rows_a, rows_b, acc_sh,
             sem_init, sem_lv, sem_sa, sem_sb, sem_a, sem_b):
    c = lax.axis_index("c")
    s = lax.axis_index("s")
    wid = s * NC + c
    ebase = wid * PER_TILE

    @pl.when((c == 0) & (s < NS - 1))
    def _init0():
        pltpu.async_copy(x_hbm.at[pl.ds(s * ROWS_PER_TILE, ROWS_PER_TILE)],
                         acc_sh.at[pl.ds(s * ROWS_PER_TILE, ROWS_PER_TILE)],
                         sem_init)

    @pl.when((c == 0) & (s == NS - 1))
    def _init0t():
        pltpu.async_copy(x_hbm.at[pl.ds(15 * ROWS_PER_TILE, XTAIL)],
                         acc_sh.at[pl.ds(15 * ROWS_PER_TILE, XTAIL)],
                         sem_init)
        pltpu.async_copy(zin_hbm.at[pl.ds(0, ZTAIL)],
                         acc_sh.at[pl.ds(N, ZTAIL)], sem_init)

    @pl.when(c == 1)
    def _init1():
        pltpu.async_copy(zin_hbm,
                         acc_sh.at[pl.ds(s * ROWS_PER_TILE, ROWS_PER_TILE)],
                         sem_init)

    pltpu.async_copy(lv_hbm, lv_v, sem_lv)
    pltpu.async_copy(ed_hbm.at[pl.ds(ebase, SB)], sed_a, sem_sa)

    pltpu.make_async_copy(lv_hbm, lv_v, sem_lv).wait()

    def compact_block(sed, cnt):
        def cvec(v, cnt):
            for u in range(SBU):
                off = (v * SBU + u) * 16
                packed = sed[pl.ds(off, 16)]
                srcs = lax.shift_right_logical(packed, 16)
                dsts = packed & 0xFFFF
                ll = plsc.load_gather(lv_v, [srcs])
                hl = plsc.load_gather(lv_v, [dsts])
                valid = (ll >= 0) & (ll < LEVELS) & (hl > ll)
                plsc.store_compressed(gcomb_v.at[pl.ds(cnt, 16)], packed,
                                      mask=valid)
                cnt = cnt + plsc.all_reduce_population_count(valid)[0]
            return cnt
        return lax.fori_loop(0, SB // (16 * SBU), cvec, cnt)

    cnt = jnp.int32(0)
    for b in range(NSB):
        cur, cur_sem = (sed_a, sem_sa) if b % 2 == 0 else (sed_b, sem_sb)
        nxt, nxt_sem = (sed_b, sem_sb) if b % 2 == 0 else (sed_a, sem_sa)
        if b + 1 < NSB:
            pltpu.async_copy(ed_hbm.at[pl.ds(ebase + (b + 1) * SB, SB)],
                             nxt, nxt_sem)
        pltpu.make_async_copy(ed_hbm.at[pl.ds(0, SB)], cur, cur_sem).wait()
        cnt = compact_block(cur, cnt)

    for v in range(C // 16):
        dummy = DUMMY0 + v * 16 + lax.iota(jnp.int32, 16)
        gcomb_v[pl.ds(cnt + v * 16, 16)] = dummy

    nch = (cnt + C - 1) // C

    @pl.when((c == 0) & (s < NS - 1))
    def _init0_wait():
        pltpu.make_async_copy(
            x_hbm.at[pl.ds(s * ROWS_PER_TILE, ROWS_PER_TILE)],
            acc_sh.at[pl.ds(s * ROWS_PER_TILE, ROWS_PER_TILE)],
            sem_init).wait()

    @pl.when((c == 0) & (s == NS - 1))
    def _init0t_wait():
        pltpu.make_async_copy(x_hbm.at[pl.ds(15 * ROWS_PER_TILE, XTAIL)],
                              acc_sh.at[pl.ds(15 * ROWS_PER_TILE, XTAIL)],
                              sem_init).wait()
        pltpu.make_async_copy(zin_hbm.at[pl.ds(0, ZTAIL)],
                              acc_sh.at[pl.ds(N, ZTAIL)], sem_init).wait()

    @pl.when(c == 1)
    def _init1_wait():
        pltpu.make_async_copy(
            zin_hbm,
            acc_sh.at[pl.ds(s * ROWS_PER_TILE, ROWS_PER_TILE)],
            sem_init).wait()

    plsc.subcore_barrier()

    def fill_idx(j, gidx, sidx):
        for v in range(C // 16):
            packed = gcomb_v[pl.ds(j * C + v * 16, 16)]
            gidx[pl.ds(v * 16, 16)] = lax.shift_right_logical(packed, 16)
            sidx[pl.ds(v * 16, 16)] = packed & 0xFFFF

    def start_gather(rows, gidx, sem):
        return pltpu.async_copy(y_hbm.at[gidx], rows, sem)

    @pl.when(nch > 0)
    def _prologue():
        fill_idx(0, gidx_a, sidx_a)
        start_gather(rows_a, gidx_a, sem_a)

    def pair(p, carry):
        j0 = 2 * p
        j1 = j0 + 1

        @pl.when(j1 < nch)
        def _startb():
            fill_idx(j1, gidx_b, sidx_b)
            start_gather(rows_b, gidx_b, sem_b)

        pltpu.make_async_copy(y_hbm.at[gidx_a], rows_a, sem_a).wait()
        pltpu.sync_copy(rows_a, acc_sh.at[sidx_a], add=True)

        @pl.when(j0 + 2 < nch)
        def _starta():
            fill_idx(j0 + 2, gidx_a, sidx_a)
            start_gather(rows_a, gidx_a, sem_a)

        @pl.when(j1 < nch)
        def _drainb():
            pltpu.make_async_copy(y_hbm.at[gidx_b], rows_b, sem_b).wait()
            pltpu.sync_copy(rows_b, acc_sh.at[sidx_b], add=True)

        return carry

    lax.fori_loop(0, (nch + 1) // 2, pair, jnp.int32(0))
    plsc.subcore_barrier()

    pltpu.sync_copy(acc_sh.at[pl.ds(s * ROWS_PER_TILE, ROWS_PER_TILE)],
                    out_hbm.at[c, pl.ds(s * ROWS_PER_TILE, ROWS_PER_TILE)])


_sc_edges = functools.partial(
    pl.kernel,
    out_type=jax.ShapeDtypeStruct((NC, ACC_N, F), jnp.float32),
    mesh=plsc.VectorSubcoreMesh(core_axis_name="c", subcore_axis_name="s"),
    compiler_params=pltpu.CompilerParams(needs_layout_passes=False),
    scratch_types=[
        pltpu.VMEM((N,), jnp.int32),
        pltpu.VMEM((SB,), jnp.int32),
        pltpu.VMEM((SB,), jnp.int32),
        pltpu.VMEM((PER_TILE + C,), jnp.int32),
        pltpu.VMEM((C,), jnp.int32),
        pltpu.VMEM((C,), jnp.int32),
        pltpu.VMEM((C,), jnp.int32),
        pltpu.VMEM((C,), jnp.int32),
        pltpu.VMEM((C, F), jnp.float32),
        pltpu.VMEM((C, F), jnp.float32),
        pltpu.VMEM_SHARED((ACC_N, F), jnp.float32),
        pltpu.SemaphoreType.DMA,
        pltpu.SemaphoreType.DMA,
        pltpu.SemaphoreType.DMA,
        pltpu.SemaphoreType.DMA,
        pltpu.SemaphoreType.DMA,
        pltpu.SemaphoreType.DMA,
    ],
)(_sc_body)


def kernel(node_features, hierarchy_edges, hierarchy_levels, level_weights, level_biases):
    packed = jnp.sum(hierarchy_edges * jnp.array([1 << 16, 1], jnp.int32),
                     axis=1, dtype=jnp.int32)

    lv3 = hierarchy_levels.reshape(NB, 1, BLK)

    y = pl.pallas_call(
        _transform_body,
        grid=(NB,),
        in_specs=[
            pl.BlockSpec((BLK, F), lambda i: (i, 0)),
            pl.BlockSpec((1, 1, BLK), lambda i: (i, 0, 0)),
            pl.BlockSpec((LEVELS, F, F), lambda i: (0, 0, 0)),
            pl.BlockSpec((LEVELS, F), lambda i: (0, 0)),
        ],
        out_specs=pl.BlockSpec((BLK, F), lambda i: (i, 0)),
        out_shape=jax.ShapeDtypeStruct((N, F), jnp.float32),
    )(node_features, lv3, level_weights, level_biases)

    zin = jnp.zeros((ROWS_PER_TILE, F), jnp.float32)
    parts = _sc_edges(y, packed, hierarchy_levels, node_features, zin)

    out = pl.pallas_call(
        _merge_body,
        grid=(NB,),
        in_specs=[
            pl.BlockSpec((NC, BLK, F), lambda i: (0, i, 0)),
        ],
        out_specs=pl.BlockSpec((BLK, F), lambda i: (i, 0)),
        out_shape=jax.ShapeDtypeStruct((N, F), jnp.float32),
    )(parts)
    return out
